# baseline (speedup 1.0000x reference)
import functools

import jax
import jax.numpy as jnp
from jax import lax
from jax.experimental import pallas as pl
from jax.experimental.pallas import tpu as pltpu
from jax.experimental.pallas import tpu_sc as plsc

_LANE = 128


@functools.lru_cache(maxsize=None)
def _make_gather(n_rows, vocab, dim):
    info = plsc.get_sparse_core_info()
    nc, ns = info.num_cores, info.num_subcores
    nw = nc * ns
    n_per_w = n_rows // nw
    k = 8
    chunk = k * _LANE
    n_chunks = n_per_w // chunk
    assert n_per_w % chunk == 0

    mesh = plsc.VectorSubcoreMesh(core_axis_name="c", subcore_axis_name="s")

    @functools.partial(
        pl.kernel,
        mesh=mesh,
        out_type=jax.ShapeDtypeStruct((n_rows, dim), jnp.float32),
        scratch_types=[
            pltpu.VMEM((k, _LANE), jnp.int32),
            pltpu.VMEM((chunk, dim), jnp.float32),
            pltpu.SemaphoreType.DMA,
        ],
        compiler_params=pltpu.CompilerParams(use_tc_tiling_on_sc=False),
    )
    def gather_kernel(ids_hbm, table_hbm, out_hbm, idx_v, rows_v, sem):
        wid = lax.axis_index("s") * nc + lax.axis_index("c")
        idx_row0 = wid * (n_per_w // _LANE)
        out_row0 = wid * n_per_w

        def chunk_body(ci, carry):
            pltpu.sync_copy(ids_hbm.at[pl.ds(idx_row0 + ci * k, k)], idx_v)
            handles = [
                pltpu.async_copy(
                    table_hbm.at[idx_v.at[j]],
                    rows_v.at[pl.ds(j * _LANE, _LANE)],
                    sem,
                )
                for j in range(k)
            ]
            for h in handles:
                h.wait()
            pltpu.sync_copy(
                rows_v, out_hbm.at[pl.ds(out_row0 + ci * chunk, chunk)]
            )
            return carry

        lax.fori_loop(0, n_chunks, chunk_body, 0)

    return gather_kernel


def kernel(input_ids, weight):
    b, s = input_ids.shape
    vocab, dim = weight.shape
    n_rows = b * s
    ids = input_ids.reshape(n_rows // _LANE, _LANE)
    out = _make_gather(n_rows, vocab, dim)(ids, weight)
    return out.reshape(b, s, dim)


# speedup vs baseline: 1.0121x; 1.0121x over previous
import functools

import jax
import jax.numpy as jnp
from jax import lax
from jax.experimental import pallas as pl
from jax.experimental.pallas import tpu as pltpu
from jax.experimental.pallas import tpu_sc as plsc

_LANE = 128


@functools.lru_cache(maxsize=None)
def _make_gather(n_rows, vocab, dim):
    info = plsc.get_sparse_core_info()
    nc, ns = info.num_cores, info.num_subcores
    nw = nc * ns
    n_per_w = n_rows // nw
    k = 4
    chunk = k * _LANE
    n_chunks = n_per_w // chunk
    idx_rows = n_per_w // _LANE
    assert n_per_w % chunk == 0 and n_chunks % 2 == 0

    mesh = plsc.VectorSubcoreMesh(core_axis_name="c", subcore_axis_name="s")

    @functools.partial(
        pl.kernel,
        mesh=mesh,
        out_type=jax.ShapeDtypeStruct((n_rows, dim), jnp.float32),
        scratch_types=[
            pltpu.VMEM((idx_rows, _LANE), jnp.int32),
            pltpu.VMEM((chunk, dim), jnp.float32),
            pltpu.VMEM((chunk, dim), jnp.float32),
            pltpu.SemaphoreType.DMA,
            pltpu.SemaphoreType.DMA,
            pltpu.SemaphoreType.DMA,
            pltpu.SemaphoreType.DMA,
        ],
        compiler_params=pltpu.CompilerParams(use_tc_tiling_on_sc=False),
    )
    def gather_kernel(ids_hbm, table_hbm, out_hbm, idx_v, rows0, rows1,
                      gs0, gs1, ws0, ws1):
        wid = lax.axis_index("s") * nc + lax.axis_index("c")
        idx_row0 = wid * idx_rows
        out_row0 = wid * n_per_w
        pltpu.sync_copy(ids_hbm.at[pl.ds(idx_row0, idx_rows)], idx_v)

        bufs = (rows0, rows1)
        gsems = (gs0, gs1)
        wsems = (ws0, ws1)

        def gfire(g, b):
            for j in range(k):
                pltpu.async_copy(
                    table_hbm.at[idx_v.at[g * k + j]],
                    bufs[b].at[pl.ds(j * _LANE, _LANE)],
                    gsems[b],
                )

        def gwait(b):
            pltpu.make_async_copy(
                out_hbm.at[pl.ds(0, chunk)], bufs[b], gsems[b]
            ).wait()

        def wfire(g, b):
            pltpu.async_copy(
                bufs[b],
                out_hbm.at[pl.ds(out_row0 + g * chunk, chunk)],
                wsems[b],
            )

        def wwait(b):
            pltpu.make_async_copy(
                bufs[b], out_hbm.at[pl.ds(0, chunk)], wsems[b]
            ).wait()

        gfire(0, 0)
        gfire(1, 1)

        def body(h, carry):
            g = 2 * h
            gwait(0)
            wfire(g, 0)
            gwait(1)
            wfire(g + 1, 1)
            wwait(0)
            gfire(g + 2, 0)
            wwait(1)
            gfire(g + 3, 1)
            return carry

        lax.fori_loop(0, n_chunks // 2 - 1, body, 0)

        g_last = n_chunks - 2
        gwait(0)
        wfire(g_last, 0)
        gwait(1)
        wfire(g_last + 1, 1)
        wwait(0)
        wwait(1)

    return gather_kernel


def kernel(input_ids, weight):
    b, s = input_ids.shape
    vocab, dim = weight.shape
    n_rows = b * s
    ids = input_ids.reshape(n_rows // _LANE, _LANE)
    out = _make_gather(n_rows, vocab, dim)(ids, weight)
    return out.reshape(b, s, dim)
